# baseline (speedup 1.0000x reference)
import functools

import jax
import jax.numpy as jnp
from jax.experimental import pallas as pl
from jax.experimental.pallas import tpu as pltpu

_B, _F, _T = 64, 128, 4096
_BT = 2048


def _body(xl_ref, fs_ref, fw_ref, ts_ref, tw_ref, x_ref, o_ref):
    b = pl.program_id(0)
    jt = pl.program_id(1)
    t0 = jt * _BT

    f_io = jax.lax.broadcasted_iota(jnp.int32, (_F, 1), 0)
    fm = jnp.zeros((_F, 1), jnp.bool_)
    for i in range(fs_ref.shape[0]):
        s = fs_ref[i]
        fm = fm | ((f_io >= s) & (f_io < s + fw_ref[i]))
    a_f = jnp.where(fm, 0.0, 1.0).astype(jnp.float32)

    t_io = jax.lax.broadcasted_iota(jnp.int32, (1, _BT), 1) + t0
    xl = xl_ref[b]
    tm = jnp.zeros((1, _BT), jnp.bool_)
    for i in range(ts_ref.shape[0]):
        s = ts_ref[i]
        tm = tm | ((t_io >= s) & (t_io < s + tw_ref[i]))
    tm = tm & (t_io < xl)
    a_t = jnp.where(tm, 0.0, 1.0).astype(jnp.float32)

    o_ref[0] = x_ref[0] * a_f * a_t


@jax.jit
def _run(x, xl, fs, fw, ts, tw):
    grid = (_B, _T // _BT)
    return pl.pallas_call(
        _body,
        grid_spec=pltpu.PrefetchScalarGridSpec(
            num_scalar_prefetch=5,
            grid=grid,
            in_specs=[
                pl.BlockSpec((1, _F, _BT), lambda b, jt, *_: (b, 0, jt)),
            ],
            out_specs=pl.BlockSpec((1, _F, _BT), lambda b, jt, *_: (b, 0, jt)),
        ),
        out_shape=jax.ShapeDtypeStruct((_B, _F, _T), jnp.float32),
        compiler_params=pltpu.CompilerParams(
            dimension_semantics=("parallel", "parallel"),
        ),
    )(xl, fs, fw, ts, tw, x)


def kernel(x, x_len, freq_starts, freq_widths, time_starts, time_widths):
    xl = x_len.astype(jnp.int32)
    fs = freq_starts.astype(jnp.int32)
    fw = freq_widths.astype(jnp.int32)
    ts = time_starts.astype(jnp.int32)
    tw = time_widths.astype(jnp.int32)
    return _run(x, xl, fs, fw, ts, tw)
